# Initial kernel scaffold; baseline (speedup 1.0000x reference)
import jax
import jax.numpy as jnp
from jax.experimental import pallas as pl


def kernel(inputs, ln_weight, ln_bias, embeddings):
    raise NotImplementedError("write your pallas kernel here")


# speedup vs baseline: 8.7476x; 8.7476x over previous
import functools

import jax
import jax.numpy as jnp
from jax.experimental import pallas as pl
from jax.experimental.pallas import tpu as pltpu

_NUM_BUCKETS = 1024
_NUM_HEADS = 4
_EMBED_DIM = 256
_HEAD_DIM = 64
_COMMITMENT_COST = 0.25
_EPSILON = 1e-5
_B, _T = 32, 1024
_N = _B * _T
_R = 512
_G = _N // _R
_KDIM = _NUM_HEADS * _NUM_BUCKETS


def _vq_kernel(x_ref, w_ref, b_ref, e_ref, e2_ref,
               out_ref, loss_ref, uniq_ref,
               counts_acc, loss_acc):
    step = pl.program_id(0)

    @pl.when(step == 0)
    def _init():
        counts_acc[...] = jnp.zeros_like(counts_acc)
        loss_acc[...] = jnp.zeros_like(loss_acc)

    x = x_ref[...]
    mu = jnp.mean(x, axis=-1, keepdims=True)
    var = jnp.mean((x - mu) ** 2, axis=-1, keepdims=True)
    x = (x - mu) / jnp.sqrt(var + 1e-5) * w_ref[...] + b_ref[...]
    x = jnp.tanh(x / 5.0) * 5.0
    n = jnp.sqrt(jnp.sum(x * x, axis=-1, keepdims=True))
    xn = x / jnp.maximum(n, _EPSILON)

    dist = jnp.dot(xn, e_ref[...], preferred_element_type=jnp.float32)

    masks = []
    for h in range(_NUM_HEADS):
        dh = dist[:, h * _NUM_BUCKETS:(h + 1) * _NUM_BUCKETS]
        mh = jnp.max(dh, axis=-1, keepdims=True)
        masks.append((dh >= mh).astype(jnp.float32))
    mask = jnp.concatenate(masks, axis=-1)

    counts_acc[...] += jnp.sum(mask, axis=0, keepdims=True)

    q = jnp.dot(mask, e2_ref[...], preferred_element_type=jnp.float32)
    out_ref[...] = q

    diff = q - xn
    loss_acc[...] += jnp.sum(diff * diff, axis=(0, 1), keepdims=True)

    @pl.when(step == _G - 1)
    def _fin():
        loss_ref[...] = (_COMMITMENT_COST / (_N * _EMBED_DIM)) * loss_acc[...]
        c = counts_acc[:, 0:_NUM_BUCKETS]
        for h in range(1, _NUM_HEADS):
            c = c + counts_acc[:, h * _NUM_BUCKETS:(h + 1) * _NUM_BUCKETS]
        uniq_ref[...] = jnp.sum((c > 0.5).astype(jnp.int32),
                                axis=(0, 1), keepdims=True)


@functools.partial(jax.jit, static_argnames=())
def kernel(inputs, ln_weight, ln_bias, embeddings):
    x = inputs.reshape(_N, _EMBED_DIM)
    w = ln_weight.reshape(1, _EMBED_DIM)
    b = ln_bias.reshape(1, _EMBED_DIM)

    eT = jnp.transpose(embeddings, (0, 2, 1))
    e_blocks = []
    e2_blocks = []
    for h in range(_NUM_HEADS):
        row = [jnp.zeros((_HEAD_DIM, _NUM_BUCKETS), jnp.float32)] * _NUM_HEADS
        row[h] = eT[h]
        e_blocks.append(jnp.concatenate(row, axis=-1))
        row2 = [jnp.zeros((_NUM_BUCKETS, _HEAD_DIM), jnp.float32)] * _NUM_HEADS
        row2[h] = embeddings[h]
        e2_blocks.append(jnp.concatenate(row2, axis=-1))
    e = jnp.concatenate(e_blocks, axis=0)
    e2 = jnp.concatenate(e2_blocks, axis=0)

    grid = (_G,)
    out, loss, uniq = pl.pallas_call(
        _vq_kernel,
        grid=grid,
        in_specs=[
            pl.BlockSpec((_R, _EMBED_DIM), lambda i: (i, 0)),
            pl.BlockSpec((1, _EMBED_DIM), lambda i: (0, 0)),
            pl.BlockSpec((1, _EMBED_DIM), lambda i: (0, 0)),
            pl.BlockSpec((_EMBED_DIM, _KDIM), lambda i: (0, 0)),
            pl.BlockSpec((_KDIM, _EMBED_DIM), lambda i: (0, 0)),
        ],
        out_specs=[
            pl.BlockSpec((_R, _EMBED_DIM), lambda i: (i, 0)),
            pl.BlockSpec((1, 1), lambda i: (0, 0)),
            pl.BlockSpec((1, 1), lambda i: (0, 0)),
        ],
        out_shape=[
            jax.ShapeDtypeStruct((_N, _EMBED_DIM), jnp.float32),
            jax.ShapeDtypeStruct((1, 1), jnp.float32),
            jax.ShapeDtypeStruct((1, 1), jnp.int32),
        ],
        scratch_shapes=[
            pltpu.VMEM((1, _KDIM), jnp.float32),
            pltpu.VMEM((1, 1), jnp.float32),
        ],
    )(x, w, b, e, e2)

    quantized_st = out.reshape(_B, _T, _EMBED_DIM)
    return (quantized_st, loss.reshape(()), uniq.reshape(()))


# speedup vs baseline: 9.5886x; 1.0961x over previous
import functools

import jax
import jax.numpy as jnp
from jax.experimental import pallas as pl
from jax.experimental.pallas import tpu as pltpu

_NUM_BUCKETS = 1024
_NUM_HEADS = 4
_EMBED_DIM = 256
_HEAD_DIM = 64
_COMMITMENT_COST = 0.25
_EPSILON = 1e-5
_B, _T = 32, 1024
_N = _B * _T
_R = 1024
_G = _N // _R
_KDIM = _NUM_HEADS * _NUM_BUCKETS


def _vq_kernel(x_ref, w_ref, b_ref, e_ref, e2_ref,
               out_ref, loss_ref, uniq_ref,
               counts_acc, loss_acc):
    step = pl.program_id(0)

    @pl.when(step == 0)
    def _init():
        counts_acc[...] = jnp.zeros_like(counts_acc)
        loss_acc[...] = jnp.zeros_like(loss_acc)

    x = x_ref[...]
    mu = jnp.mean(x, axis=-1, keepdims=True)
    var = jnp.mean((x - mu) ** 2, axis=-1, keepdims=True)
    x = (x - mu) / jnp.sqrt(var + 1e-5) * w_ref[...] + b_ref[...]
    x = jnp.tanh(x / 5.0) * 5.0
    n = jnp.sqrt(jnp.sum(x * x, axis=-1, keepdims=True))
    xn = x / jnp.maximum(n, _EPSILON)

    dist = jnp.dot(xn, e_ref[...], preferred_element_type=jnp.float32)

    masks = []
    for h in range(_NUM_HEADS):
        dh = dist[:, h * _NUM_BUCKETS:(h + 1) * _NUM_BUCKETS]
        mh = jnp.max(dh, axis=-1, keepdims=True)
        masks.append((dh >= mh).astype(jnp.float32))
    mask = jnp.concatenate(masks, axis=-1)

    counts_acc[...] += jnp.sum(mask, axis=0, keepdims=True)

    q = jnp.dot(mask, e2_ref[...], preferred_element_type=jnp.float32)
    out_ref[...] = q

    diff = q - xn
    loss_acc[...] += jnp.sum(diff * diff, axis=(0, 1), keepdims=True)

    @pl.when(step == _G - 1)
    def _fin():
        loss_ref[...] = (_COMMITMENT_COST / (_N * _EMBED_DIM)) * loss_acc[...]
        c = counts_acc[0:1, 0:_NUM_BUCKETS]
        for h in range(1, _NUM_HEADS):
            c = c + counts_acc[0:1, h * _NUM_BUCKETS:(h + 1) * _NUM_BUCKETS]
        uniq_ref[...] = jnp.sum((c > 0.5).astype(jnp.int32),
                                axis=(0, 1), keepdims=True)


@functools.partial(jax.jit, static_argnames=())
def kernel(inputs, ln_weight, ln_bias, embeddings):
    x = inputs.reshape(_N, _EMBED_DIM)
    w = ln_weight.reshape(1, _EMBED_DIM)
    b = ln_bias.reshape(1, _EMBED_DIM)

    eT = jnp.transpose(embeddings, (0, 2, 1))
    e_blocks = []
    e2_blocks = []
    for h in range(_NUM_HEADS):
        row = [jnp.zeros((_HEAD_DIM, _NUM_BUCKETS), jnp.float32)] * _NUM_HEADS
        row[h] = eT[h]
        e_blocks.append(jnp.concatenate(row, axis=-1))
        row2 = [jnp.zeros((_NUM_BUCKETS, _HEAD_DIM), jnp.float32)] * _NUM_HEADS
        row2[h] = embeddings[h]
        e2_blocks.append(jnp.concatenate(row2, axis=-1))
    e = jnp.concatenate(e_blocks, axis=0)
    e2 = jnp.concatenate(e2_blocks, axis=0)

    grid = (_G,)
    out, loss, uniq = pl.pallas_call(
        _vq_kernel,
        grid=grid,
        in_specs=[
            pl.BlockSpec((_R, _EMBED_DIM), lambda i: (i, 0)),
            pl.BlockSpec((1, _EMBED_DIM), lambda i: (0, 0)),
            pl.BlockSpec((1, _EMBED_DIM), lambda i: (0, 0)),
            pl.BlockSpec((_EMBED_DIM, _KDIM), lambda i: (0, 0)),
            pl.BlockSpec((_KDIM, _EMBED_DIM), lambda i: (0, 0)),
        ],
        out_specs=[
            pl.BlockSpec((_R, _EMBED_DIM), lambda i: (i, 0)),
            pl.BlockSpec((1, 1), lambda i: (0, 0)),
            pl.BlockSpec((1, 1), lambda i: (0, 0)),
        ],
        out_shape=[
            jax.ShapeDtypeStruct((_N, _EMBED_DIM), jnp.float32),
            jax.ShapeDtypeStruct((1, 1), jnp.float32),
            jax.ShapeDtypeStruct((1, 1), jnp.int32),
        ],
        scratch_shapes=[
            pltpu.VMEM((1, _KDIM), jnp.float32),
            pltpu.VMEM((1, 1), jnp.float32),
        ],
    )(x, w, b, e, e2)

    quantized_st = out.reshape(_B, _T, _EMBED_DIM)
    return (quantized_st, loss.reshape(()), uniq.reshape(()))
